# baseline (speedup 1.0000x reference)
import jax
import jax.numpy as jnp
from jax import lax
from jax.experimental import pallas as pl
from jax.experimental.pallas import tpu as pltpu
from jax.experimental.pallas import tpu_sc as plsc

DIM = 128
NFLOW = 6
NTRUST = 6
NTAB = NFLOW * NTRUST
NPAIR = NTAB * NTAB
NW = 32
CHUNKP = 32
NSLOT = 4


def _fuse_pairs_body(flow_ref, trust_ref, out_ref):
    rows36 = lax.broadcasted_iota(jnp.int32, (NTAB, NFLOW), 0)
    cols6 = lax.broadcasted_iota(jnp.int32, (NTAB, NFLOW), 1)
    pick_flow = (rows36 // NTRUST == cols6).astype(jnp.float32)
    pick_trust = (rows36 % NTRUST == cols6).astype(jnp.float32)
    left36 = jnp.dot(pick_flow, flow_ref[...], preferred_element_type=jnp.float32)
    right36 = jnp.dot(pick_trust, trust_ref[...], preferred_element_type=jnp.float32)
    t36 = jnp.concatenate([left36, right36], axis=1)

    rowsp = lax.broadcasted_iota(jnp.int32, (NPAIR, NTAB), 0)
    cols36 = lax.broadcasted_iota(jnp.int32, (NPAIR, NTAB), 1)
    pick_hi = (rowsp // NTAB == cols36).astype(jnp.float32)
    pick_lo = (rowsp % NTAB == cols36).astype(jnp.float32)
    hi = jnp.dot(pick_hi, t36, preferred_element_type=jnp.float32)
    lo = jnp.dot(pick_lo, t36, preferred_element_type=jnp.float32)
    out_ref[...] = jnp.concatenate([hi, lo], axis=1)


def _fuse_pairs(flow_table, trust_table):
    return pl.pallas_call(
        _fuse_pairs_body,
        out_shape=jax.ShapeDtypeStruct((NPAIR, 4 * DIM), jnp.float32),
    )(flow_table, trust_table)


def _sc_lookup_body(fe_hbm, fo_hbm, te_hbm, to_hbm, tab_hbm, out_hbm, *scratch):
    slots = tuple(scratch[i * 9:(i + 1) * 9] for i in range(NSLOT))
    npair_total = out_hbm.shape[0]
    per = npair_total // NW
    nchunk = per // CHUNKP
    wid = lax.axis_index("s") * 2 + lax.axis_index("c")
    base = wid * per

    def off_of(g):
        return pl.multiple_of(base + g * CHUNKP, CHUNKP)

    def start_idx(g, slot):
        fe_v, fo_v, te_v, to_v, _, _, isem, _, _ = slot
        off = off_of(g)
        pltpu.async_copy(fe_hbm.at[pl.ds(off, CHUNKP)], fe_v, isem)
        pltpu.async_copy(fo_hbm.at[pl.ds(off, CHUNKP)], fo_v, isem)
        pltpu.async_copy(te_hbm.at[pl.ds(off, CHUNKP)], te_v, isem)
        pltpu.async_copy(to_hbm.at[pl.ds(off, CHUNKP)], to_v, isem)

    def wait_idx(g, slot):
        fe_v, fo_v, te_v, to_v, _, _, isem, _, _ = slot
        off = off_of(g)
        pltpu.make_async_copy(fe_hbm.at[pl.ds(off, CHUNKP)], fe_v, isem).wait()
        pltpu.make_async_copy(fo_hbm.at[pl.ds(off, CHUNKP)], fo_v, isem).wait()
        pltpu.make_async_copy(te_hbm.at[pl.ds(off, CHUNKP)], te_v, isem).wait()
        pltpu.make_async_copy(to_hbm.at[pl.ds(off, CHUNKP)], to_v, isem).wait()

    def compute_p(slot):
        fe_v, fo_v, te_v, to_v, p_v = slot[0], slot[1], slot[2], slot[3], slot[4]
        for j in range(CHUNKP // 16):
            sl = pl.ds(j * 16, 16)
            c_e = fe_v[sl] * NTRUST + te_v[sl]
            c_o = fo_v[sl] * NTRUST + to_v[sl]
            p_v[sl] = c_e * NTAB + c_o

    def start_gather(slot):
        p_v, rows, gsem = slot[4], slot[5], slot[8]
        pltpu.async_copy(tab_hbm.at[p_v], rows, gsem)

    def wait_gather(slot):
        p_v, rows, gsem = slot[4], slot[5], slot[8]
        pltpu.make_async_copy(tab_hbm.at[p_v], rows, gsem).wait()

    def start_out(g, slot):
        rows, osem = slot[5], slot[7]
        pltpu.async_copy(rows, out_hbm.at[pl.ds(off_of(g), CHUNKP)], osem)

    def wait_out(g, slot):
        rows, osem = slot[5], slot[7]
        pltpu.make_async_copy(rows, out_hbm.at[pl.ds(off_of(g), CHUNKP)], osem).wait()

    for s in range(NSLOT):
        start_idx(s, slots[s])

    def body(i, _):
        for b in range(NSLOT):
            g = i * NSLOT + b
            slot = slots[b]
            wait_idx(g, slot)
            compute_p(slot)

            @pl.when(g >= NSLOT)
            def _():
                wait_out(g - NSLOT, slot)

            start_gather(slot)

            @pl.when(g + NSLOT < nchunk)
            def _():
                start_idx(g + NSLOT, slot)

            prev = slots[(b + 1) % NSLOT]

            @pl.when(g >= NSLOT - 1)
            def _():
                wait_gather(prev)
                start_out(g - (NSLOT - 1), prev)
        return 0

    lax.fori_loop(0, nchunk // NSLOT, body, 0)

    for g in range(nchunk - (NSLOT - 1), nchunk):
        slot = slots[g % NSLOT]
        wait_gather(slot)
        start_out(g, slot)
    for g in range(nchunk - NSLOT, nchunk):
        wait_out(g, slots[g % NSLOT])


def _sc_lookup(fe, fo, te, to, tab):
    npair_total = fe.shape[0]
    slot_scratch = []
    for _ in range(NSLOT):
        slot_scratch += [
            pltpu.VMEM((CHUNKP,), jnp.int32),
            pltpu.VMEM((CHUNKP,), jnp.int32),
            pltpu.VMEM((CHUNKP,), jnp.int32),
            pltpu.VMEM((CHUNKP,), jnp.int32),
            pltpu.VMEM((CHUNKP,), jnp.int32),
            pltpu.VMEM((CHUNKP, 4 * DIM), jnp.float32),
            pltpu.SemaphoreType.DMA,
            pltpu.SemaphoreType.DMA,
            pltpu.SemaphoreType.DMA,
        ]
    run = pl.kernel(
        _sc_lookup_body,
        out_type=jax.ShapeDtypeStruct((npair_total, 4 * DIM), jnp.float32),
        mesh=plsc.VectorSubcoreMesh(core_axis_name="c", subcore_axis_name="s"),
        scratch_types=slot_scratch,
    )
    return run(fe, fo, te, to, tab)


def kernel(data_flows, trust_levels, flow_table, trust_table):
    b, e = data_flows.shape
    tab = _fuse_pairs(flow_table, trust_table)
    fp = data_flows.reshape(-1, 2)
    tp = trust_levels.reshape(-1, 2)
    out = _sc_lookup(fp[:, 0], fp[:, 1], tp[:, 0], tp[:, 1], tab)
    return out.reshape(b, e, 2 * DIM)


# speedup vs baseline: 1.9038x; 1.9038x over previous
import jax
import jax.numpy as jnp
from jax import lax
from jax.experimental import pallas as pl
from jax.experimental.pallas import tpu as pltpu
from jax.experimental.pallas import tpu_sc as plsc

DIM = 128
NFLOW = 6
NTRUST = 6
NTAB = NFLOW * NTRUST
NW = 32
CHUNK = 80
NSLOT = 4
REP = 32
TC_ROWS = 122880
TC_BLOCK = 2048


def _fuse_tables_body(flow_ref, trust_ref, out_ref):
    nrow = REP * NTAB
    rows = lax.broadcasted_iota(jnp.int32, (nrow, NFLOW), 0) % NTAB
    cols = lax.broadcasted_iota(jnp.int32, (nrow, NFLOW), 1)
    pick_flow = (rows // NTRUST == cols).astype(jnp.float32)
    pick_trust = (rows % NTRUST == cols).astype(jnp.float32)
    left = jnp.dot(pick_flow, flow_ref[...], preferred_element_type=jnp.float32)
    right = jnp.dot(pick_trust, trust_ref[...], preferred_element_type=jnp.float32)
    out_ref[...] = jnp.concatenate([left, right], axis=1)


def _fuse_tables(flow_table, trust_table):
    return pl.pallas_call(
        _fuse_tables_body,
        out_shape=jax.ShapeDtypeStruct((REP * NTAB, 2 * DIM), jnp.float32),
    )(flow_table, trust_table)


def _tc_expand_body(f_ref, t_ref, flow_ref, trust_ref, out_ref):
    f = f_ref[...]
    t = t_ref[...]
    k = lax.broadcasted_iota(jnp.int32, (TC_BLOCK, NFLOW), 1)
    onehot_f = (f[:, None] == k).astype(jnp.float32)
    onehot_t = (t[:, None] == k).astype(jnp.float32)
    left = jnp.dot(onehot_f, flow_ref[...], preferred_element_type=jnp.float32)
    right = jnp.dot(onehot_t, trust_ref[...], preferred_element_type=jnp.float32)
    out_ref[...] = jnp.concatenate([left, right], axis=1)


def _tc_expand(f_idx, t_idx, flow_table, trust_table):
    n = f_idx.shape[0]
    grid = (n // TC_BLOCK,)
    return pl.pallas_call(
        _tc_expand_body,
        grid=grid,
        in_specs=[
            pl.BlockSpec((TC_BLOCK,), lambda i: (i,)),
            pl.BlockSpec((TC_BLOCK,), lambda i: (i,)),
            pl.BlockSpec((NFLOW, DIM), lambda i: (0, 0)),
            pl.BlockSpec((NTRUST, DIM), lambda i: (0, 0)),
        ],
        out_specs=pl.BlockSpec((TC_BLOCK, 2 * DIM), lambda i: (i, 0)),
        out_shape=jax.ShapeDtypeStruct((n, 2 * DIM), jnp.float32),
    )(f_idx, t_idx, flow_table, trust_table)


def _sc_lookup_body(f_hbm, t_hbm, tab_hbm, out_hbm, *scratch):
    slots = tuple(scratch[i * 7:(i + 1) * 7] for i in range(NSLOT))
    n = out_hbm.shape[0]
    per = n // NW
    nchunk = per // CHUNK
    wid = lax.axis_index("s") * 2 + lax.axis_index("c")
    base = wid * per

    def off_of(g):
        return pl.multiple_of(base + g * CHUNK, CHUNK)

    def start_idx(g, slot):
        f_v, t_v, _, _, isem, _, _ = slot
        off = off_of(g)
        pltpu.async_copy(f_hbm.at[pl.ds(off, CHUNK)], f_v, isem)
        pltpu.async_copy(t_hbm.at[pl.ds(off, CHUNK)], t_v, isem)

    def wait_idx(g, slot):
        f_v, t_v, _, _, isem, _, _ = slot
        off = off_of(g)
        pltpu.make_async_copy(f_hbm.at[pl.ds(off, CHUNK)], f_v, isem).wait()
        pltpu.make_async_copy(t_hbm.at[pl.ds(off, CHUNK)], t_v, isem).wait()

    def compute_c(slot):
        f_v, t_v, c_v = slot[0], slot[1], slot[2]
        rep_base = (wid % REP) * NTAB
        for j in range(CHUNK // 16):
            sl = pl.ds(j * 16, 16)
            c_v[sl] = f_v[sl] * NTRUST + t_v[sl] + rep_base

    def start_gather(slot):
        c_v, rows, gsem = slot[2], slot[3], slot[6]
        pltpu.async_copy(tab_hbm.at[c_v], rows, gsem)

    def wait_gather(slot):
        c_v, rows, gsem = slot[2], slot[3], slot[6]
        pltpu.make_async_copy(tab_hbm.at[c_v], rows, gsem).wait()

    def start_out(g, slot):
        rows, osem = slot[3], slot[5]
        pltpu.async_copy(rows, out_hbm.at[pl.ds(off_of(g), CHUNK)], osem)

    def wait_out(g, slot):
        rows, osem = slot[3], slot[5]
        pltpu.make_async_copy(rows, out_hbm.at[pl.ds(off_of(g), CHUNK)], osem).wait()

    for s in range(NSLOT):
        start_idx(s, slots[s])

    def body(i, _):
        for b in range(NSLOT):
            g = i * NSLOT + b
            slot = slots[b]
            wait_idx(g, slot)
            compute_c(slot)

            @pl.when(g >= NSLOT)
            def _():
                wait_out(g - NSLOT, slot)

            start_gather(slot)

            @pl.when(g + NSLOT < nchunk)
            def _():
                start_idx(g + NSLOT, slot)

            prev = slots[(b + 1) % NSLOT]

            @pl.when(g >= NSLOT - 1)
            def _():
                wait_gather(prev)
                start_out(g - (NSLOT - 1), prev)
        return 0

    lax.fori_loop(0, nchunk // NSLOT, body, 0)

    for g in range(nchunk - (NSLOT - 1), nchunk):
        slot = slots[g % NSLOT]
        wait_gather(slot)
        start_out(g, slot)
    for g in range(nchunk - NSLOT, nchunk):
        wait_out(g, slots[g % NSLOT])


def _sc_lookup(f_idx, t_idx, tab):
    n = f_idx.shape[0]
    slot_scratch = []
    for _ in range(NSLOT):
        slot_scratch += [
            pltpu.VMEM((CHUNK,), jnp.int32),
            pltpu.VMEM((CHUNK,), jnp.int32),
            pltpu.VMEM((CHUNK,), jnp.int32),
            pltpu.VMEM((CHUNK, 2 * DIM), jnp.float32),
            pltpu.SemaphoreType.DMA,
            pltpu.SemaphoreType.DMA,
            pltpu.SemaphoreType.DMA,
        ]
    run = pl.kernel(
        _sc_lookup_body,
        out_type=jax.ShapeDtypeStruct((n, 2 * DIM), jnp.float32),
        mesh=plsc.VectorSubcoreMesh(core_axis_name="c", subcore_axis_name="s"),
        scratch_types=slot_scratch,
    )
    return run(f_idx, t_idx, tab)


def kernel(data_flows, trust_levels, flow_table, trust_table):
    b, e = data_flows.shape
    f = data_flows.reshape(-1)
    t = trust_levels.reshape(-1)
    tab = _fuse_tables(flow_table, trust_table)
    sc_out = _sc_lookup(f[TC_ROWS:], t[TC_ROWS:], tab)
    tc_out = _tc_expand(f[:TC_ROWS], t[:TC_ROWS], flow_table, trust_table)
    out = jnp.concatenate([tc_out, sc_out], axis=0)
    return out.reshape(b, e, 2 * DIM)


# speedup vs baseline: 3.3071x; 1.7371x over previous
import jax
import jax.numpy as jnp
from jax import lax
from jax.experimental import pallas as pl
from jax.experimental.pallas import tpu as pltpu
from jax.experimental.pallas import tpu_sc as plsc

DIM = 128
NFLOW = 6
NTRUST = 6
NTAB = NFLOW * NTRUST
NW = 32
CHUNK = 80
NSLOT = 4
REP = 32
TC_ROWS = 143360
TC_BLOCK = 2048


def _fuse_tables_body(flow_ref, trust_ref, out_ref):
    nrow = REP * NTAB
    rows = lax.broadcasted_iota(jnp.int32, (nrow, NFLOW), 0) % NTAB
    cols = lax.broadcasted_iota(jnp.int32, (nrow, NFLOW), 1)
    pick_flow = (rows // NTRUST == cols).astype(jnp.float32)
    pick_trust = (rows % NTRUST == cols).astype(jnp.float32)
    left = jnp.dot(pick_flow, flow_ref[...], preferred_element_type=jnp.float32)
    right = jnp.dot(pick_trust, trust_ref[...], preferred_element_type=jnp.float32)
    out_ref[...] = jnp.concatenate([left, right], axis=1)


def _fuse_tables(flow_table, trust_table):
    return pl.pallas_call(
        _fuse_tables_body,
        out_shape=jax.ShapeDtypeStruct((REP * NTAB, 2 * DIM), jnp.float32),
    )(flow_table, trust_table)


def _tc_expand_body(f_ref, t_ref, flow_ref, trust_ref, out_ref):
    f = f_ref[...]
    t = t_ref[...]
    k = lax.broadcasted_iota(jnp.int32, (TC_BLOCK, NFLOW), 1)
    onehot_f = (f[:, None] == k).astype(jnp.float32)
    onehot_t = (t[:, None] == k).astype(jnp.float32)
    left = jnp.dot(onehot_f, flow_ref[...], preferred_element_type=jnp.float32)
    right = jnp.dot(onehot_t, trust_ref[...], preferred_element_type=jnp.float32)
    out_ref[...] = jnp.concatenate([left, right], axis=1)


def _tc_expand(f_idx, t_idx, flow_table, trust_table, total_n):
    grid = (TC_ROWS // TC_BLOCK,)
    return pl.pallas_call(
        _tc_expand_body,
        grid=grid,
        in_specs=[
            pl.BlockSpec((TC_BLOCK,), lambda i: (i,)),
            pl.BlockSpec((TC_BLOCK,), lambda i: (i,)),
            pl.BlockSpec((NFLOW, DIM), lambda i: (0, 0)),
            pl.BlockSpec((NTRUST, DIM), lambda i: (0, 0)),
        ],
        out_specs=pl.BlockSpec((TC_BLOCK, 2 * DIM), lambda i: (i, 0)),
        out_shape=jax.ShapeDtypeStruct((total_n, 2 * DIM), jnp.float32),
    )(f_idx, t_idx, flow_table, trust_table)


def _sc_lookup_body(f_hbm, t_hbm, tab_hbm, out_hbm, *scratch):
    slots = tuple(scratch[i * 7:(i + 1) * 7] for i in range(NSLOT))
    n = out_hbm.shape[0] - TC_ROWS
    per = n // NW
    nchunk = per // CHUNK
    wid = lax.axis_index("s") * 2 + lax.axis_index("c")
    base = wid * per

    def off_of(g):
        return pl.multiple_of(TC_ROWS + base + g * CHUNK, CHUNK)

    def ioff_of(g):
        return pl.multiple_of(base + g * CHUNK, CHUNK)

    def start_idx(g, slot):
        f_v, t_v, _, _, isem, _, _ = slot
        off = ioff_of(g)
        pltpu.async_copy(f_hbm.at[pl.ds(off, CHUNK)], f_v, isem)
        pltpu.async_copy(t_hbm.at[pl.ds(off, CHUNK)], t_v, isem)

    def wait_idx(g, slot):
        f_v, t_v, _, _, isem, _, _ = slot
        off = ioff_of(g)
        pltpu.make_async_copy(f_hbm.at[pl.ds(off, CHUNK)], f_v, isem).wait()
        pltpu.make_async_copy(t_hbm.at[pl.ds(off, CHUNK)], t_v, isem).wait()

    def compute_c(slot):
        f_v, t_v, c_v = slot[0], slot[1], slot[2]
        rep_base = (wid % REP) * NTAB
        for j in range(CHUNK // 16):
            sl = pl.ds(j * 16, 16)
            c_v[sl] = f_v[sl] * NTRUST + t_v[sl] + rep_base

    def start_gather(slot):
        c_v, rows, gsem = slot[2], slot[3], slot[6]
        pltpu.async_copy(tab_hbm.at[c_v], rows, gsem)

    def wait_gather(slot):
        c_v, rows, gsem = slot[2], slot[3], slot[6]
        pltpu.make_async_copy(tab_hbm.at[c_v], rows, gsem).wait()

    def start_out(g, slot):
        rows, osem = slot[3], slot[5]
        pltpu.async_copy(rows, out_hbm.at[pl.ds(off_of(g), CHUNK)], osem)

    def wait_out(g, slot):
        rows, osem = slot[3], slot[5]
        pltpu.make_async_copy(rows, out_hbm.at[pl.ds(off_of(g), CHUNK)], osem).wait()

    for s in range(NSLOT):
        start_idx(s, slots[s])

    def body(i, _):
        for b in range(NSLOT):
            g = i * NSLOT + b
            slot = slots[b]
            wait_idx(g, slot)
            compute_c(slot)

            @pl.when(g >= NSLOT)
            def _():
                wait_out(g - NSLOT, slot)

            start_gather(slot)

            @pl.when(g + NSLOT < nchunk)
            def _():
                start_idx(g + NSLOT, slot)

            prev = slots[(b + 1) % NSLOT]

            @pl.when(g >= NSLOT - 1)
            def _():
                wait_gather(prev)
                start_out(g - (NSLOT - 1), prev)
        return 0

    lax.fori_loop(0, nchunk // NSLOT, body, 0)

    for g in range(nchunk - (NSLOT - 1), nchunk):
        slot = slots[g % NSLOT]
        wait_gather(slot)
        start_out(g, slot)
    for g in range(nchunk - NSLOT, nchunk):
        wait_out(g, slots[g % NSLOT])


def _sc_lookup(f_idx, t_idx, tab, buf_ref):
    slot_scratch = []
    for _ in range(NSLOT):
        slot_scratch += [
            pltpu.VMEM((CHUNK,), jnp.int32),
            pltpu.VMEM((CHUNK,), jnp.int32),
            pltpu.VMEM((CHUNK,), jnp.int32),
            pltpu.VMEM((CHUNK, 2 * DIM), jnp.float32),
            pltpu.SemaphoreType.DMA,
            pltpu.SemaphoreType.DMA,
            pltpu.SemaphoreType.DMA,
        ]
    run = pl.kernel(
        _sc_lookup_body,
        out_type=(),
        mesh=plsc.VectorSubcoreMesh(core_axis_name="c", subcore_axis_name="s"),
        scratch_types=slot_scratch,
    )
    run(f_idx, t_idx, tab, buf_ref)


def kernel(data_flows, trust_levels, flow_table, trust_table):
    b, e = data_flows.shape
    n = b * e
    f = data_flows.reshape(-1)
    t = trust_levels.reshape(-1)
    tab = _fuse_tables(flow_table, trust_table)
    buf = _tc_expand(f[:TC_ROWS], t[:TC_ROWS], flow_table, trust_table, n)
    buf_ref = jax.new_ref(buf)
    _sc_lookup(f[TC_ROWS:], t[TC_ROWS:], tab, buf_ref)
    return buf_ref[...].reshape(b, e, 2 * DIM)
